# baseline (speedup 1.0000x reference)
import jax
import jax.numpy as jnp
from jax import lax
from jax.experimental import pallas as pl
from jax.experimental.pallas import tpu as pltpu
from jax.experimental.pallas import tpu_sc as plsc

NC, NS, L = 2, 16, 16
NW = NC * NS
HID = 128
ROWS = 65
NF = 4
B, S = 4096, 200
BG = B // 128
NBLK = S * BG
BPW = NBLK // NW
SLAB = NF * 128
TAB_WORDS = NF * ROWS * HID


def _body(ids_hbm, tab_hbm, out_hbm, tab_v, ids_v, out_v, sem_tab, sem_ids,
          sem_out):
    wid = lax.axis_index("s") * NC + lax.axis_index("c")
    k0 = wid * BPW

    pltpu.async_copy(tab_hbm, tab_v, sem_tab).wait()

    def load_ids(g, slot):
        return pltpu.async_copy(
            ids_hbm.at[pl.ds((k0 + g) * SLAB, SLAB)],
            ids_v.at[pl.ds(slot * SLAB, SLAB)], sem_ids)

    def store_out(g, slot):
        k = k0 + g
        s = k // BG
        bg = lax.rem(k, BG)
        return pltpu.async_copy(
            out_v.at[slot], out_hbm.at[pl.ds(bg * 128, 128), s], sem_out)

    load_ids(0, 0).wait()

    def slab_body(g, _):
        slot = lax.rem(g, 2)

        @pl.when(g + 1 < BPW)
        def _():
            load_ids(g + 1, 1 - slot)

        @plsc.parallel_loop(0, 128 // L, unroll=2)
        def tok_body(q):
            vecs = [
                ids_v[pl.ds(slot * SLAB + f * 128 + q * L, L)]
                for f in range(NF)
            ]
            for j in range(L):
                o0 = vecs[0][j]
                o1 = vecs[1][j]
                o2 = vecs[2][j]
                o3 = vecs[3][j]
                for c in range(HID // L):
                    t0 = tab_v[pl.ds(o0 + c * L, L)]
                    t1 = tab_v[pl.ds(o1 + c * L, L)]
                    t2 = tab_v[pl.ds(o2 + c * L, L)]
                    t3 = tab_v[pl.ds(o3 + c * L, L)]
                    out_v[slot, q * L + j, pl.ds(c * L, L)] = \
                        (t0 + t1) + (t2 + t3)

        @pl.when(g >= 2)
        def _():
            pltpu.make_async_copy(out_v.at[0],
                                  out_hbm.at[pl.ds(0, 128), 0],
                                  sem_out).wait()

        store_out(g, slot)

        @pl.when(g + 1 < BPW)
        def _():
            pltpu.make_async_copy(
                ids_v.at[pl.ds(0, SLAB)],
                ids_hbm.at[pl.ds(0, SLAB)], sem_ids).wait()
        return 0

    lax.fori_loop(0, BPW, slab_body, 0)

    for _ in range(2):
        pltpu.make_async_copy(out_v.at[0], out_hbm.at[pl.ds(0, 128), 0],
                              sem_out).wait()


@jax.jit
def _run(offs_flat, tab_flat):
    mesh = plsc.VectorSubcoreMesh(core_axis_name="c", subcore_axis_name="s",
                                  num_cores=NC, num_subcores=NS)
    return pl.kernel(
        _body,
        out_type=jax.ShapeDtypeStruct((B, S, HID), jnp.float32),
        mesh=mesh,
        scratch_types=[
            pltpu.VMEM((TAB_WORDS,), jnp.float32),
            pltpu.VMEM((2 * SLAB,), jnp.int32),
            pltpu.VMEM((2, 128, HID), jnp.float32),
            pltpu.SemaphoreType.DMA,
            pltpu.SemaphoreType.DMA,
            pltpu.SemaphoreType.DMA,
        ],
        compiler_params=pltpu.CompilerParams(needs_layout_passes=False),
    )(offs_flat, tab_flat)


def kernel(noise_ids, W0, W1, W2, W3):
    featbase = jnp.array([i * ROWS * HID for i in range(NF)], jnp.int32)
    offs = noise_ids * HID + featbase
    offs_flat = (offs.reshape(BG, 128, S, NF)
                 .transpose(2, 0, 3, 1)
                 .reshape(B * S * NF))
    tab_flat = jnp.concatenate([W0, W1, W2, W3], axis=0).reshape(-1)
    return _run(offs_flat, tab_flat)
